# Initial kernel scaffold; baseline (speedup 1.0000x reference)
import jax
import jax.numpy as jnp
from jax.experimental import pallas as pl


def kernel(x, labels, weight, bias):
    raise NotImplementedError("write your pallas kernel here")


# speedup vs baseline: 1.0005x; 1.0005x over previous
import functools

import jax
import jax.numpy as jnp
from jax.experimental import pallas as pl
from jax.experimental.pallas import tpu as pltpu

_N_CLASSES = 8
_EPS = 1e-5


def _one_hot_f32(lab, n):
    iota = jax.lax.broadcasted_iota(jnp.int32, (lab.shape[0], n), 1)
    return (lab == iota).astype(jnp.float32)


def _stats_kernel(lab_ref, x_ref, sum_ref, sum2_ref, cnt_ref):
    b = pl.program_id(1)

    @pl.when(b == 0)
    def _():
        sum_ref[...] = jnp.zeros_like(sum_ref)
        sum2_ref[...] = jnp.zeros_like(sum2_ref)
        cnt_ref[...] = jnp.zeros_like(cnt_ref)

    xb = x_ref[...]
    s = jnp.sum(xb, axis=2)
    s2 = jnp.sum(xb * xb, axis=2)
    m = _one_hot_f32(lab_ref[...], _N_CLASSES)
    dn = (((0,), (0,)), ((), ()))
    sum_ref[...] += jax.lax.dot_general(s, m, dn, preferred_element_type=jnp.float32)
    sum2_ref[...] += jax.lax.dot_general(s2, m, dn, preferred_element_type=jnp.float32)
    cnt_ref[...] += jnp.broadcast_to(
        jnp.sum(m, axis=0, keepdims=True), cnt_ref.shape)


def _apply_kernel(lab_ref, x_ref, sum_ref, sum2_ref, cnt_ref, w_ref, b_ref,
                  o_ref, *, ell):
    cnt = jnp.maximum(cnt_ref[0:1, :] * ell, 1.0)
    mean = sum_ref[...] / cnt
    var = sum2_ref[...] / cnt - mean * mean
    inv = jax.lax.rsqrt(var + _EPS)
    sc = inv * w_ref[...]
    sh = b_ref[...] - mean * sc
    m = _one_hot_f32(lab_ref[...], _N_CLASSES)
    dn = (((1,), (1,)), ((), ()))
    row_sc = jax.lax.dot_general(m, sc, dn, preferred_element_type=jnp.float32)
    row_sh = jax.lax.dot_general(m, sh, dn, preferred_element_type=jnp.float32)
    o_ref[...] = x_ref[...] * row_sc[:, :, None] + row_sh[:, :, None]


def kernel(x, labels, weight, bias):
    B, F, L = x.shape
    K = weight.shape[0]
    lab2d = labels.reshape(B, 1)
    w_t = weight.T
    b_t = bias.T

    bb_s, fb_s = 64, 64
    nf, nb = F // fb_s, B // bb_s
    sums, sums2, cnt_raw = pl.pallas_call(
        _stats_kernel,
        grid=(nf, nb),
        in_specs=[
            pl.BlockSpec((bb_s, 1), lambda f, b: (b, 0)),
            pl.BlockSpec((bb_s, fb_s, L), lambda f, b: (b, f, 0)),
        ],
        out_specs=[
            pl.BlockSpec((fb_s, K), lambda f, b: (f, 0)),
            pl.BlockSpec((fb_s, K), lambda f, b: (f, 0)),
            pl.BlockSpec((8, K), lambda f, b: (f, 0)),
        ],
        out_shape=[
            jax.ShapeDtypeStruct((F, K), jnp.float32),
            jax.ShapeDtypeStruct((F, K), jnp.float32),
            jax.ShapeDtypeStruct((8 * nf, K), jnp.float32),
        ],
        compiler_params=pltpu.CompilerParams(
            dimension_semantics=("parallel", "arbitrary"),
            vmem_limit_bytes=50 * 1024 * 1024,
        ),
        name="cbn_stats",
    )(lab2d, x)

    bb_a, fb_a = 32, 64
    nf2, nb2 = F // fb_a, B // bb_a
    out = pl.pallas_call(
        functools.partial(_apply_kernel, ell=float(L)),
        grid=(nf2, nb2),
        in_specs=[
            pl.BlockSpec((bb_a, 1), lambda f, b: (b, 0)),
            pl.BlockSpec((bb_a, fb_a, L), lambda f, b: (b, f, 0)),
            pl.BlockSpec((fb_a, K), lambda f, b: (f, 0)),
            pl.BlockSpec((fb_a, K), lambda f, b: (f, 0)),
            pl.BlockSpec((8, K), lambda f, b: (0, 0)),
            pl.BlockSpec((fb_a, K), lambda f, b: (f, 0)),
            pl.BlockSpec((fb_a, K), lambda f, b: (f, 0)),
        ],
        out_specs=pl.BlockSpec((bb_a, fb_a, L), lambda f, b: (b, f, 0)),
        out_shape=jax.ShapeDtypeStruct((B, F, L), jnp.float32),
        compiler_params=pltpu.CompilerParams(
            dimension_semantics=("parallel", "arbitrary"),
            vmem_limit_bytes=50 * 1024 * 1024,
        ),
        name="cbn_apply",
    )(lab2d, x, sums, sums2, cnt_raw, w_t, b_t)
    return out


# speedup vs baseline: 1.0035x; 1.0030x over previous
import functools

import jax
import jax.numpy as jnp
from jax.experimental import pallas as pl
from jax.experimental.pallas import tpu as pltpu

_N_CLASSES = 8
_EPS = 1e-5


def _one_hot_f32(lab, n):
    iota = jax.lax.broadcasted_iota(jnp.int32, (lab.shape[0], n), 1)
    return (lab == iota).astype(jnp.float32)


def _fused_kernel(lab_ref, x_ref, w_ref, b_ref, o_ref,
                  sum_ref, sum2_ref, cnt_ref, *, ell):
    p = pl.program_id(1)
    b = pl.program_id(2)

    @pl.when((p == 0) & (b == 0))
    def _():
        sum_ref[...] = jnp.zeros_like(sum_ref)
        sum2_ref[...] = jnp.zeros_like(sum2_ref)
        cnt_ref[...] = jnp.zeros_like(cnt_ref)

    @pl.when(p == 0)
    def _():
        xb = x_ref[...]
        s = jnp.sum(xb, axis=2)
        s2 = jnp.sum(xb * xb, axis=2)
        m = _one_hot_f32(lab_ref[...], _N_CLASSES)
        dn = (((0,), (0,)), ((), ()))
        sum_ref[...] += jax.lax.dot_general(
            s, m, dn, preferred_element_type=jnp.float32)
        sum2_ref[...] += jax.lax.dot_general(
            s2, m, dn, preferred_element_type=jnp.float32)
        cnt_ref[...] += jnp.sum(m, axis=0, keepdims=True)

    @pl.when(p == 1)
    def _():
        cnt = jnp.maximum(cnt_ref[...] * ell, 1.0)
        mean = sum_ref[...] / cnt
        var = sum2_ref[...] / cnt - mean * mean
        inv = jax.lax.rsqrt(var + _EPS)
        sc = inv * w_ref[...]
        sh = b_ref[...] - mean * sc
        m = _one_hot_f32(lab_ref[...], _N_CLASSES)
        dn = (((1,), (1,)), ((), ()))
        row_sc = jax.lax.dot_general(
            m, sc, dn, preferred_element_type=jnp.float32)
        row_sh = jax.lax.dot_general(
            m, sh, dn, preferred_element_type=jnp.float32)
        o_ref[...] = x_ref[...] * row_sc[:, :, None] + row_sh[:, :, None]


def kernel(x, labels, weight, bias):
    B, F, L = x.shape
    K = weight.shape[0]
    lab2d = labels.reshape(B, 1)
    w_t = weight.T
    b_t = bias.T

    bb, fb = 32, 64
    nf, nb = F // fb, B // bb
    out = pl.pallas_call(
        functools.partial(_fused_kernel, ell=float(L)),
        grid=(nf, 2, nb),
        in_specs=[
            pl.BlockSpec((bb, 1), lambda f, p, b: (b, 0)),
            pl.BlockSpec((bb, fb, L), lambda f, p, b: (b, f, 0)),
            pl.BlockSpec((fb, K), lambda f, p, b: (f, 0)),
            pl.BlockSpec((fb, K), lambda f, p, b: (f, 0)),
        ],
        out_specs=pl.BlockSpec(
            (bb, fb, L), lambda f, p, b: (jnp.where(p == 0, 0, b), f, 0)),
        out_shape=jax.ShapeDtypeStruct((B, F, L), jnp.float32),
        scratch_shapes=[
            pltpu.VMEM((fb, K), jnp.float32),
            pltpu.VMEM((fb, K), jnp.float32),
            pltpu.VMEM((1, K), jnp.float32),
        ],
        compiler_params=pltpu.CompilerParams(
            dimension_semantics=("parallel", "arbitrary", "arbitrary"),
            vmem_limit_bytes=50 * 1024 * 1024,
        ),
        name="cbn_fused",
    )(lab2d, x, w_t, b_t)
    return out
